# baseline (speedup 1.0000x reference)
import functools

import jax
import jax.numpy as jnp
from jax import lax
from jax.experimental import pallas as pl
from jax.experimental.pallas import tpu as pltpu
from jax.experimental.pallas import tpu_sc as plsc

N = 10000
E = 320000
D = 128

NC = 2
NS = 16
NW = NC * NS
CH = 128
EPW = E // NW
EPWP = 10240
CPW = EPWP // CH
NST = 2
SPC = CPW // NST
SEDG = SPC * CH
NP = 10112
RPT = NP // NS
TRASH = NP - 1

_mesh = plsc.VectorSubcoreMesh(core_axis_name="c", subcore_axis_name="s")


@functools.partial(
    pl.kernel,
    mesh=_mesh,
    out_type=jax.ShapeDtypeStruct((NC, NP, D), jnp.float32),
    scratch_types=(
        pltpu.VMEM((SEDG,), jnp.int32),
        pltpu.VMEM((SPC, CH), jnp.int32),
        pltpu.VMEM((CH, D), jnp.float32),
        pltpu.VMEM((CH, D), jnp.float32),
        pltpu.VMEM_SHARED((NP, D), jnp.float32),
        pltpu.SemaphoreType.DMA,
        pltpu.SemaphoreType.DMA,
        pltpu.SemaphoreType.DMA,
        pltpu.SemaphoreType.DMA,
    ),
)
def _sc_agg(x_hbm, col_hbm, row_hbm, z128_hbm,
            agg_hbm,
            colv, rowv, rows_a, rows_b, agg_sh,
            gsem_a, gsem_b, ssem_a, ssem_b):
    cid = lax.axis_index("c")
    sid = lax.axis_index("s")
    wid = sid * NC + cid
    pltpu.sync_copy(z128_hbm, agg_sh.at[pl.ds(sid * RPT, RPT)])
    plsc.subcore_barrier()

    def cidx(k):
        return colv.at[pl.ds(pl.multiple_of(k * CH, CH), CH)]

    def gwait(buf, sem):
        pltpu.make_async_copy(x_hbm.at[cidx(0)], buf, sem).wait()

    def sstart(buf, k, sem):
        pltpu.async_copy(buf, agg_sh.at[rowv.at[k]], sem, add=True)

    def swait(buf, sem):
        pltpu.make_async_copy(buf, agg_sh.at[rowv.at[0]], sem).wait()

    def half(h, c):
        pltpu.sync_copy(col_hbm.at[wid, h], colv)
        pltpu.sync_copy(row_hbm.at[wid, h], rowv)
        pltpu.async_copy(x_hbm.at[cidx(0)], rows_a, gsem_a)
        pltpu.async_copy(x_hbm.at[cidx(1)], rows_b, gsem_b)

        def pair(j, cc):
            k = 2 * j
            gwait(rows_a, gsem_a)
            sstart(rows_a, k, ssem_a)
            gwait(rows_b, gsem_b)
            sstart(rows_b, k + 1, ssem_b)
            swait(rows_a, ssem_a)
            pltpu.async_copy(x_hbm.at[cidx(k + 2)], rows_a, gsem_a)
            swait(rows_b, ssem_b)
            pltpu.async_copy(x_hbm.at[cidx(k + 3)], rows_b, gsem_b)
            return cc

        lax.fori_loop(0, (SPC - 2) // 2, pair, 0)
        gwait(rows_a, gsem_a)
        sstart(rows_a, SPC - 2, ssem_a)
        gwait(rows_b, gsem_b)
        sstart(rows_b, SPC - 1, ssem_b)
        swait(rows_a, ssem_a)
        swait(rows_b, ssem_b)
        return c

    lax.fori_loop(0, NST, half, 0)

    plsc.subcore_barrier()
    sl = pl.ds(sid * RPT, RPT)
    pltpu.sync_copy(agg_sh.at[sl], agg_hbm.at[cid].at[sl])


@functools.partial(
    pl.kernel,
    mesh=_mesh,
    out_type=jax.ShapeDtypeStruct((NC, NP, D), jnp.float32),
    scratch_types=(
        pltpu.VMEM((CPW, CH), jnp.int32),
        pltpu.VMEM((CH, D), jnp.float32),
        pltpu.VMEM_SHARED((NP, D), jnp.float32),
        pltpu.SemaphoreType.DMA,
    ),
)
def _sc_deg(ones_hbm, row_hbm, z128_hbm,
            deg_hbm,
            rowv, onesv, deg_sh, ssem):
    cid = lax.axis_index("c")
    sid = lax.axis_index("s")
    wid = sid * NC + cid
    pltpu.sync_copy(z128_hbm, deg_sh.at[pl.ds(sid * RPT, RPT)])
    pltpu.sync_copy(ones_hbm, onesv)
    pltpu.sync_copy(row_hbm.at[wid], rowv)
    plsc.subcore_barrier()

    def dwait():
        pltpu.make_async_copy(onesv, deg_sh.at[rowv.at[0]], ssem).wait()

    pltpu.async_copy(onesv, deg_sh.at[rowv.at[0]], ssem, add=True)
    pltpu.async_copy(onesv, deg_sh.at[rowv.at[1]], ssem, add=True)

    def body(k, c):
        pltpu.async_copy(onesv, deg_sh.at[rowv.at[k]], ssem, add=True)
        dwait()
        return c

    lax.fori_loop(2, CPW, body, 0)
    dwait()
    dwait()

    plsc.subcore_barrier()
    sl = pl.ds(sid * RPT, RPT)
    pltpu.sync_copy(deg_sh.at[sl], deg_hbm.at[cid].at[sl])


BLK = 2000


def _tc_body(agg_ref, deg_ref, x_ref, w_ref, out_ref):
    a = agg_ref[0] + agg_ref[1]
    d = deg_ref[0, :, 0:1] + deg_ref[1, :, 0:1]
    scale = 1.0 / jnp.maximum(d, 1.0)
    y = lax.dot_general(a, w_ref[...], (((1,), (1,)), ((), ())),
                        preferred_element_type=jnp.float32)
    out_ref[...] = y * scale + x_ref[...]


def _tc_body_final(agg_ref, deg_ref, x_ref, w_ref, res_ref, out_ref):
    a = agg_ref[0] + agg_ref[1]
    d = deg_ref[0, :, 0:1] + deg_ref[1, :, 0:1]
    scale = 1.0 / jnp.maximum(d, 1.0)
    y = lax.dot_general(a, w_ref[...], (((1,), (1,)), ((), ())),
                        preferred_element_type=jnp.float32)
    out_ref[...] = y * scale + x_ref[...] + res_ref[...]


_AGG_SPEC = pl.BlockSpec((NC, BLK, D), lambda i: (0, i, 0))
_X_SPEC = pl.BlockSpec((BLK, D), lambda i: (i, 0))
_W_SPEC = pl.BlockSpec((D, D), lambda i: (0, 0))


def _tc_layer(agg, deg, x, w):
    return pl.pallas_call(
        _tc_body,
        grid=(N // BLK,),
        in_specs=[_AGG_SPEC, _AGG_SPEC, _X_SPEC, _W_SPEC],
        out_specs=_X_SPEC,
        out_shape=jax.ShapeDtypeStruct((N, D), jnp.float32),
    )(agg, deg, x, w)


def _tc_layer_final(agg, deg, x, w, res):
    return pl.pallas_call(
        _tc_body_final,
        grid=(N // BLK,),
        in_specs=[_AGG_SPEC, _AGG_SPEC, _X_SPEC, _W_SPEC, _X_SPEC],
        out_specs=_X_SPEC,
        out_shape=jax.ShapeDtypeStruct((N, D), jnp.float32),
    )(agg, deg, x, w, res)


def kernel(x, edge_index, ring_polarities, W0, W1, W2):
    del ring_polarities
    pad = EPWP - EPW
    row_p = jnp.pad(edge_index[0].reshape(NW, EPW), ((0, 0), (0, pad)),
                    constant_values=TRASH)
    col_p = jnp.pad(edge_index[1].reshape(NW, EPW), ((0, 0), (0, pad)))
    row2 = row_p.reshape(NW, NST, SPC, CH)
    col2 = col_p.reshape(NW, NST, SEDG)
    rowd = row_p.reshape(NW, CPW, CH)
    z128 = jnp.zeros((RPT, D), jnp.float32)
    ones128 = jnp.ones((CH, D), jnp.float32)

    deg = _sc_deg(ones128, rowd, z128)
    agg0 = _sc_agg(x, col2, row2, z128)
    x1 = _tc_layer(agg0, deg, x, W0)
    agg1 = _sc_agg(x1, col2, row2, z128)
    x2 = _tc_layer(agg1, deg, x1, W1)
    agg2 = _sc_agg(x2, col2, row2, z128)
    return _tc_layer_final(agg2, deg, x2, W2, x)


# speedup vs baseline: 1.0038x; 1.0038x over previous
import functools

import jax
import jax.numpy as jnp
from jax import lax
from jax.experimental import pallas as pl
from jax.experimental.pallas import tpu as pltpu
from jax.experimental.pallas import tpu_sc as plsc

N = 10000
E = 320000
D = 128

NC = 2
NS = 16
NW = NC * NS
CH = 128
EPW = E // NW
EPWP = 10240
CPW = EPWP // CH
NST = 2
SPC = CPW // NST
SEDG = SPC * CH
NP = 10112
RPT = NP // NS
TRASH = NP - 1

_mesh = plsc.VectorSubcoreMesh(core_axis_name="c", subcore_axis_name="s")


@functools.partial(
    pl.kernel,
    mesh=_mesh,
    out_type=jax.ShapeDtypeStruct((NC, NP, D), jnp.float32),
    scratch_types=(
        pltpu.VMEM((SEDG,), jnp.int32),
        pltpu.VMEM((SPC, CH), jnp.int32),
        pltpu.VMEM((CH, D), jnp.float32),
        pltpu.VMEM((CH, D), jnp.float32),
        pltpu.VMEM_SHARED((NP, D), jnp.float32),
        pltpu.SemaphoreType.DMA,
        pltpu.SemaphoreType.DMA,
        pltpu.SemaphoreType.DMA,
        pltpu.SemaphoreType.DMA,
    ),
)
def _sc_agg(x_hbm, col_hbm, row_hbm, z128_hbm,
            agg_hbm,
            colv, rowv, rows_a, rows_b, agg_sh,
            gsem_a, gsem_b, ssem_a, ssem_b):
    cid = lax.axis_index("c")
    sid = lax.axis_index("s")
    wid = sid * NC + cid
    pltpu.sync_copy(z128_hbm, agg_sh.at[pl.ds(sid * RPT, RPT)])
    plsc.subcore_barrier()

    def cidx(k):
        return colv.at[pl.ds(pl.multiple_of(k * CH, CH), CH)]

    def gwait(buf, sem):
        pltpu.make_async_copy(x_hbm.at[cidx(0)], buf, sem).wait()

    def sstart(buf, k, sem):
        pltpu.async_copy(buf, agg_sh.at[rowv.at[k]], sem, add=True)

    def swait(buf, sem):
        pltpu.make_async_copy(buf, agg_sh.at[rowv.at[0]], sem).wait()

    def half(h, c):
        pltpu.sync_copy(col_hbm.at[wid, h], colv)
        pltpu.sync_copy(row_hbm.at[wid, h], rowv)
        pltpu.async_copy(x_hbm.at[cidx(0)], rows_a, gsem_a)
        pltpu.async_copy(x_hbm.at[cidx(1)], rows_b, gsem_b)

        def pair(j, cc):
            k = 2 * j
            gwait(rows_a, gsem_a)
            sstart(rows_a, k, ssem_a)
            gwait(rows_b, gsem_b)
            sstart(rows_b, k + 1, ssem_b)
            swait(rows_a, ssem_a)
            pltpu.async_copy(x_hbm.at[cidx(k + 2)], rows_a, gsem_a)
            swait(rows_b, ssem_b)
            pltpu.async_copy(x_hbm.at[cidx(k + 3)], rows_b, gsem_b)
            return cc

        lax.fori_loop(0, (SPC - 2) // 2, pair, 0)
        gwait(rows_a, gsem_a)
        sstart(rows_a, SPC - 2, ssem_a)
        gwait(rows_b, gsem_b)
        sstart(rows_b, SPC - 1, ssem_b)
        swait(rows_a, ssem_a)
        swait(rows_b, ssem_b)
        return c

    lax.fori_loop(0, NST, half, 0)

    plsc.subcore_barrier()
    sl = pl.ds(sid * RPT, RPT)
    pltpu.sync_copy(agg_sh.at[sl], agg_hbm.at[cid].at[sl])


@functools.partial(
    pl.kernel,
    mesh=_mesh,
    out_type=jax.ShapeDtypeStruct((NC, NP, D), jnp.float32),
    scratch_types=(
        pltpu.VMEM((CPW, CH), jnp.int32),
        pltpu.VMEM((CH, D), jnp.float32),
        pltpu.VMEM_SHARED((NP, D), jnp.float32),
        pltpu.SemaphoreType.DMA,
    ),
)
def _sc_deg(ones_hbm, row_hbm, z128_hbm,
            deg_hbm,
            rowv, onesv, deg_sh, ssem):
    cid = lax.axis_index("c")
    sid = lax.axis_index("s")
    wid = sid * NC + cid
    pltpu.sync_copy(z128_hbm, deg_sh.at[pl.ds(sid * RPT, RPT)])
    pltpu.sync_copy(ones_hbm, onesv)
    pltpu.sync_copy(row_hbm.at[wid], rowv)
    plsc.subcore_barrier()

    def dwait():
        pltpu.make_async_copy(onesv, deg_sh.at[rowv.at[0]], ssem).wait()

    pltpu.async_copy(onesv, deg_sh.at[rowv.at[0]], ssem, add=True)
    pltpu.async_copy(onesv, deg_sh.at[rowv.at[1]], ssem, add=True)

    def body(k, c):
        pltpu.async_copy(onesv, deg_sh.at[rowv.at[k]], ssem, add=True)
        dwait()
        return c

    lax.fori_loop(2, CPW, body, 0)
    dwait()
    dwait()

    plsc.subcore_barrier()
    sl = pl.ds(sid * RPT, RPT)
    pltpu.sync_copy(deg_sh.at[sl], deg_hbm.at[cid].at[sl])


BLK = 2000


def _tc_body(agg_ref, deg_ref, x_ref, w_ref, out_ref):
    a = agg_ref[0] + agg_ref[1]
    d = deg_ref[0, :, 0:1] + deg_ref[1, :, 0:1]
    scale = 1.0 / jnp.maximum(d, 1.0)
    y = lax.dot_general(a, w_ref[...], (((1,), (1,)), ((), ())),
                        preferred_element_type=jnp.float32)
    out_ref[...] = y * scale + x_ref[...]


def _tc_body_final(agg_ref, deg_ref, x_ref, w_ref, res_ref, out_ref):
    a = agg_ref[0] + agg_ref[1]
    d = deg_ref[0, :, 0:1] + deg_ref[1, :, 0:1]
    scale = 1.0 / jnp.maximum(d, 1.0)
    y = lax.dot_general(a, w_ref[...], (((1,), (1,)), ((), ())),
                        preferred_element_type=jnp.float32)
    out_ref[...] = y * scale + x_ref[...] + res_ref[...]


_AGG_SPEC = pl.BlockSpec((NC, BLK, D), lambda i: (0, i, 0))
_X_SPEC = pl.BlockSpec((BLK, D), lambda i: (i, 0))
_W_SPEC = pl.BlockSpec((D, D), lambda i: (0, 0))


def _tc_layer(agg, deg, x, w):
    return pl.pallas_call(
        _tc_body,
        grid=(N // BLK,),
        in_specs=[_AGG_SPEC, _AGG_SPEC, _X_SPEC, _W_SPEC],
        out_specs=_X_SPEC,
        out_shape=jax.ShapeDtypeStruct((N, D), jnp.float32),
    )(agg, deg, x, w)


def _tc_layer_final(agg, deg, x, w, res):
    return pl.pallas_call(
        _tc_body_final,
        grid=(N // BLK,),
        in_specs=[_AGG_SPEC, _AGG_SPEC, _X_SPEC, _W_SPEC, _X_SPEC],
        out_specs=_X_SPEC,
        out_shape=jax.ShapeDtypeStruct((N, D), jnp.float32),
    )(agg, deg, x, w, res)


def kernel(x, edge_index, ring_polarities, W0, W1, W2):
    del ring_polarities
    pad = EPWP - EPW
    trash = jnp.broadcast_to(N + (jnp.arange(pad) % (NP - N)), (NW, pad))
    row_p = jnp.concatenate(
        [edge_index[0].reshape(NW, EPW), trash.astype(jnp.int32)], axis=1)
    col_p = jnp.pad(edge_index[1].reshape(NW, EPW), ((0, 0), (0, pad)))
    row2 = row_p.reshape(NW, NST, SPC, CH)
    col2 = col_p.reshape(NW, NST, SEDG)
    rowd = row_p.reshape(NW, CPW, CH)
    z128 = jnp.zeros((RPT, D), jnp.float32)
    ones128 = jnp.ones((CH, D), jnp.float32)

    deg = _sc_deg(ones128, rowd, z128)
    agg0 = _sc_agg(x, col2, row2, z128)
    x1 = _tc_layer(agg0, deg, x, W0)
    agg1 = _sc_agg(x1, col2, row2, z128)
    x2 = _tc_layer(agg1, deg, x1, W1)
    agg2 = _sc_agg(x2, col2, row2, z128)
    return _tc_layer_final(agg2, deg, x2, W2, x)


# speedup vs baseline: 2.3788x; 2.3697x over previous
import functools

import jax
import jax.numpy as jnp
from jax import lax
from jax.experimental import pallas as pl
from jax.experimental.pallas import tpu as pltpu
from jax.experimental.pallas import tpu_sc as plsc

N = 10000
E = 320000
D = 128

NC = 2
NS = 16
NW = NC * NS
CH = 80
EPW = E // NW
NCHUNK = EPW // CH
NP = 10112
RPT = NP // NS

_mesh = plsc.VectorSubcoreMesh(core_axis_name="c", subcore_axis_name="s")


@functools.partial(
    pl.kernel,
    mesh=_mesh,
    out_type=jax.ShapeDtypeStruct((NC, NP, D), jnp.float32),
    scratch_types=(
        pltpu.VMEM((EPW,), jnp.int32),
        pltpu.VMEM((NCHUNK, CH), jnp.int32),
        pltpu.VMEM((CH, D), jnp.float32),
        pltpu.VMEM((CH, D), jnp.float32),
        pltpu.VMEM_SHARED((NP, D), jnp.float32),
        pltpu.SemaphoreType.DMA,
        pltpu.SemaphoreType.DMA,
        pltpu.SemaphoreType.DMA,
        pltpu.SemaphoreType.DMA,
    ),
)
def _sc_agg(x_hbm, col_hbm, row_hbm, z128_hbm,
            agg_hbm,
            colv, rowv, rows_a, rows_b, agg_sh,
            gsem_a, gsem_b, ssem_a, ssem_b):
    cid = lax.axis_index("c")
    sid = lax.axis_index("s")
    wid = sid * NC + cid
    pltpu.sync_copy(z128_hbm, agg_sh.at[pl.ds(sid * RPT, RPT)])
    pltpu.sync_copy(col_hbm.at[wid, 0], colv)
    pltpu.sync_copy(row_hbm.at[wid], rowv)
    plsc.subcore_barrier()

    def cidx(k):
        return colv.at[pl.ds(pl.multiple_of(k * CH, CH), CH)]

    def gwait(buf, sem):
        pltpu.make_async_copy(x_hbm.at[cidx(0)], buf, sem).wait()

    def sstart(buf, k, sem):
        pltpu.async_copy(buf, agg_sh.at[rowv.at[k]], sem, add=True)

    def swait(buf, sem):
        pltpu.make_async_copy(buf, agg_sh.at[rowv.at[0]], sem).wait()

    pltpu.async_copy(x_hbm.at[cidx(0)], rows_a, gsem_a)
    pltpu.async_copy(x_hbm.at[cidx(1)], rows_b, gsem_b)

    def pair(j, c):
        k = 2 * j
        gwait(rows_a, gsem_a)
        sstart(rows_a, k, ssem_a)
        gwait(rows_b, gsem_b)
        sstart(rows_b, k + 1, ssem_b)
        swait(rows_a, ssem_a)
        pltpu.async_copy(x_hbm.at[cidx(k + 2)], rows_a, gsem_a)
        swait(rows_b, ssem_b)
        pltpu.async_copy(x_hbm.at[cidx(k + 3)], rows_b, gsem_b)
        return c

    lax.fori_loop(0, (NCHUNK - 3) // 2, pair, 0)
    gwait(rows_a, gsem_a)
    sstart(rows_a, NCHUNK - 3, ssem_a)
    gwait(rows_b, gsem_b)
    sstart(rows_b, NCHUNK - 2, ssem_b)
    swait(rows_a, ssem_a)
    pltpu.async_copy(x_hbm.at[cidx(NCHUNK - 1)], rows_a, gsem_a)
    gwait(rows_a, gsem_a)
    sstart(rows_a, NCHUNK - 1, ssem_a)
    swait(rows_b, ssem_b)
    swait(rows_a, ssem_a)

    plsc.subcore_barrier()
    sl = pl.ds(sid * RPT, RPT)
    pltpu.sync_copy(agg_sh.at[sl], agg_hbm.at[cid].at[sl])


@functools.partial(
    pl.kernel,
    mesh=_mesh,
    out_type=jax.ShapeDtypeStruct((NC, NP, D), jnp.float32),
    scratch_types=(
        pltpu.VMEM((NCHUNK, CH), jnp.int32),
        pltpu.VMEM((CH, D), jnp.float32),
        pltpu.VMEM_SHARED((NP, D), jnp.float32),
        pltpu.SemaphoreType.DMA,
    ),
)
def _sc_deg(ones_hbm, row_hbm, z128_hbm,
            deg_hbm,
            rowv, onesv, deg_sh, ssem):
    cid = lax.axis_index("c")
    sid = lax.axis_index("s")
    wid = sid * NC + cid
    pltpu.sync_copy(z128_hbm, deg_sh.at[pl.ds(sid * RPT, RPT)])
    pltpu.sync_copy(ones_hbm, onesv)
    pltpu.sync_copy(row_hbm.at[wid], rowv)
    plsc.subcore_barrier()

    def dwait():
        pltpu.make_async_copy(onesv, deg_sh.at[rowv.at[0]], ssem).wait()

    pltpu.async_copy(onesv, deg_sh.at[rowv.at[0]], ssem, add=True)
    pltpu.async_copy(onesv, deg_sh.at[rowv.at[1]], ssem, add=True)

    def body(k, c):
        pltpu.async_copy(onesv, deg_sh.at[rowv.at[k]], ssem, add=True)
        dwait()
        return c

    lax.fori_loop(2, NCHUNK, body, 0)
    dwait()
    dwait()

    plsc.subcore_barrier()
    sl = pl.ds(sid * RPT, RPT)
    pltpu.sync_copy(deg_sh.at[sl], deg_hbm.at[cid].at[sl])


BLK = 2000


def _tc_body(agg_ref, deg_ref, x_ref, w_ref, out_ref):
    a = agg_ref[0] + agg_ref[1]
    d = deg_ref[0, :, 0:1] + deg_ref[1, :, 0:1]
    scale = 1.0 / jnp.maximum(d, 1.0)
    y = lax.dot_general(a, w_ref[...], (((1,), (1,)), ((), ())),
                        preferred_element_type=jnp.float32)
    out_ref[...] = y * scale + x_ref[...]


def _tc_body_final(agg_ref, deg_ref, x_ref, w_ref, res_ref, out_ref):
    a = agg_ref[0] + agg_ref[1]
    d = deg_ref[0, :, 0:1] + deg_ref[1, :, 0:1]
    scale = 1.0 / jnp.maximum(d, 1.0)
    y = lax.dot_general(a, w_ref[...], (((1,), (1,)), ((), ())),
                        preferred_element_type=jnp.float32)
    out_ref[...] = y * scale + x_ref[...] + res_ref[...]


_AGG_SPEC = pl.BlockSpec((NC, BLK, D), lambda i: (0, i, 0))
_X_SPEC = pl.BlockSpec((BLK, D), lambda i: (i, 0))
_W_SPEC = pl.BlockSpec((D, D), lambda i: (0, 0))


def _tc_layer(agg, deg, x, w):
    return pl.pallas_call(
        _tc_body,
        grid=(N // BLK,),
        in_specs=[_AGG_SPEC, _AGG_SPEC, _X_SPEC, _W_SPEC],
        out_specs=_X_SPEC,
        out_shape=jax.ShapeDtypeStruct((N, D), jnp.float32),
    )(agg, deg, x, w)


def _tc_layer_final(agg, deg, x, w, res):
    return pl.pallas_call(
        _tc_body_final,
        grid=(N // BLK,),
        in_specs=[_AGG_SPEC, _AGG_SPEC, _X_SPEC, _W_SPEC, _X_SPEC],
        out_specs=_X_SPEC,
        out_shape=jax.ShapeDtypeStruct((N, D), jnp.float32),
    )(agg, deg, x, w, res)


def kernel(x, edge_index, ring_polarities, W0, W1, W2):
    del ring_polarities
    row2 = edge_index[0].reshape(NW, NCHUNK, CH)
    col2 = edge_index[1].reshape(NW, 1, EPW)
    z128 = jnp.zeros((RPT, D), jnp.float32)
    ones128 = jnp.ones((CH, D), jnp.float32)

    deg = _sc_deg(ones128, row2, z128)
    agg0 = _sc_agg(x, col2, row2, z128)
    x1 = _tc_layer(agg0, deg, x, W0)
    agg1 = _sc_agg(x1, col2, row2, z128)
    x2 = _tc_layer(agg1, deg, x1, W1)
    agg2 = _sc_agg(x2, col2, row2, z128)
    return _tc_layer_final(agg2, deg, x2, W2, x)
